# Initial kernel scaffold; baseline (speedup 1.0000x reference)
import jax
import jax.numpy as jnp
from jax.experimental import pallas as pl


def kernel(rel_emb_sr, rel_emb_tg, conf_sr, imp_sr, pca_sr, conf_tg, imp_tg, pca_tg, head_sr, tail_sr, relation_sr, head_tg, tail_tg, relation_tg):
    raise NotImplementedError("write your pallas kernel here")


# speedup vs baseline: 183.0860x; 183.0860x over previous
import functools

import jax
import jax.numpy as jnp
from jax import lax
from jax.experimental import pallas as pl
from jax.experimental.pallas import tpu as pltpu
from jax.experimental.pallas import tpu_sc as plsc

N = 50000
E = 1600000
NC = 2
NS = 16
NT = NC * NS
EPT = E // NT
B = 2000
NCH = EPT // B
NPAD = 50176
SLICE = NPAD // NS
DIAG_T = 25
DIAG_B = N // DIAG_T

@functools.cache
def _mesh():
    return plsc.VectorSubcoreMesh(
        core_axis_name="c", subcore_axis_name="s",
        num_cores=NC, num_subcores=NS)


_f32 = jnp.float32
_i32 = jnp.int32


def _rsqrt_newton(x):
    i = plsc.bitcast(x, _i32)
    y = plsc.bitcast(jnp.int32(0x5F3759DF) - (i >> 1), _f32)
    for _ in range(3):
        y = y * (1.5 - 0.5 * x * y * y)
    return jnp.where(x > 0.0, y, 0.0)


def _k1_graph(conf_h, imp_h, pca_h, head_h, vals_h, part_h,
              acc, cb, ib, pb, hb, vb, sem, wid):
    base = wid * EPT

    def zero_body(i, carry):
        acc[pl.ds(i * 16, 16)] = jnp.zeros((16,), _f32)
        return carry
    lax.fori_loop(0, NPAD // 16, zero_body, 0)

    def chunk_body(k, carry):
        off = base + k * B
        d1 = pltpu.async_copy(conf_h.at[pl.ds(off, B)], cb, sem)
        d2 = pltpu.async_copy(imp_h.at[pl.ds(off, B)], ib, sem)
        d3 = pltpu.async_copy(pca_h.at[pl.ds(off, B)], pb, sem)
        d4 = pltpu.async_copy(head_h.at[pl.ds(off, B)], hb, sem)
        d1.wait(); d2.wait(); d3.wait(); d4.wait()

        def vec_body(j, c2):
            s = pl.ds(j * 16, 16)
            v = cb[s] * ib[s] * pb[s]
            vb[s] = v
            plsc.addupdate_scatter(acc, [hb[s]], v)
            return c2
        lax.fori_loop(0, B // 16, vec_body, 0)
        pltpu.sync_copy(vb, vals_h.at[pl.ds(off, B)])
        return carry
    lax.fori_loop(0, NCH, chunk_body, 0)
    pltpu.sync_copy(acc, part_h.at[pl.ds(wid * NPAD, NPAD)])


def _k1_body(conf_sr, imp_sr, pca_sr, head_sr,
             conf_tg, imp_tg, pca_tg, head_tg,
             vals_sr, vals_tg, part_sr, part_tg,
             acc, cb, ib, pb, hb, vb, sem):
    wid = lax.axis_index("s") * NC + lax.axis_index("c")
    _k1_graph(conf_sr, imp_sr, pca_sr, head_sr, vals_sr, part_sr,
              acc, cb, ib, pb, hb, vb, sem, wid)
    _k1_graph(conf_tg, imp_tg, pca_tg, head_tg, vals_tg, part_tg,
              acc, cb, ib, pb, hb, vb, sem, wid)


@functools.cache
def _k1():
    return functools.partial(
        pl.kernel,
        out_type=(
            jax.ShapeDtypeStruct((E,), _f32),
            jax.ShapeDtypeStruct((E,), _f32),
            jax.ShapeDtypeStruct((NT * NPAD,), _f32),
            jax.ShapeDtypeStruct((NT * NPAD,), _f32),
        ),
        mesh=_mesh(),
        compiler_params=pltpu.CompilerParams(needs_layout_passes=False),
        scratch_types=(
            pltpu.VMEM((NPAD,), _f32),
            pltpu.VMEM((B,), _f32),
            pltpu.VMEM((B,), _f32),
            pltpu.VMEM((B,), _f32),
            pltpu.VMEM((B,), _i32),
            pltpu.VMEM((B,), _f32),
            pltpu.SemaphoreType.DMA,
        ),
    )(_k1_body)


def _k2_graph(head_h, tail_h, vals_h, part_h, adj_h,
              d_sh, d_ref, rs, rowbuf, hb, tb, vb, ob, sem, cid, sid, wid):
    soff = sid * SLICE

    def one_body(i, carry):
        rs[pl.ds(i * 16, 16)] = jnp.full((16,), 1.0, _f32)
        return carry
    lax.fori_loop(0, SLICE // 16, one_body, 0)

    def row_body(r, carry):
        pltpu.sync_copy(part_h.at[pl.ds(r * NPAD + soff, SLICE)], rowbuf)

        def add_body(j, c2):
            s = pl.ds(j * 16, 16)
            rs[s] = rs[s] + rowbuf[s]
            return c2
        lax.fori_loop(0, SLICE // 16, add_body, 0)
        return carry
    lax.fori_loop(0, NT, row_body, 0)

    def newton_body(j, carry):
        s = pl.ds(j * 16, 16)
        rowbuf[s] = _rsqrt_newton(rs[s])
        return carry
    lax.fori_loop(0, SLICE // 16, newton_body, 0)
    pltpu.sync_copy(rowbuf, d_sh.at[pl.ds(soff, SLICE)])
    plsc.subcore_barrier()

    pltpu.sync_copy(d_sh, d_ref)

    base = wid * EPT

    def chunk_body(k, carry):
        off = base + k * B
        d1 = pltpu.async_copy(head_h.at[pl.ds(off, B)], hb, sem)
        d2 = pltpu.async_copy(tail_h.at[pl.ds(off, B)], tb, sem)
        d3 = pltpu.async_copy(vals_h.at[pl.ds(off, B)], vb, sem)
        d1.wait(); d2.wait(); d3.wait()

        def vec_body(j, c2):
            s = pl.ds(j * 16, 16)
            dh = plsc.load_gather(d_ref, [hb[s]])
            dt = plsc.load_gather(d_ref, [tb[s]])
            ob[s] = vb[s] * dh * dt
            return c2
        lax.fori_loop(0, B // 16, vec_body, 0)
        pltpu.sync_copy(ob, adj_h.at[pl.ds(off, B)])
        return carry
    lax.fori_loop(0, NCH, chunk_body, 0)

    @pl.when(wid < DIAG_T)
    def _():
        doff = wid * DIAG_B

        def diag_body(j, carry):
            s = pl.ds(j * 16, 16)
            y = d_ref[pl.ds(doff + j * 16, 16)]
            ob[s] = y * y
            return carry
        lax.fori_loop(0, DIAG_B // 16, diag_body, 0)
        pltpu.sync_copy(ob, adj_h.at[pl.ds(E + doff, DIAG_B)])


def _k2_body(head_sr, tail_sr, vals_sr, part_sr,
             head_tg, tail_tg, vals_tg, part_tg,
             adj_sr, adj_tg,
             d_sh, d_ref, rs, rowbuf, hb, tb, vb, ob, sem):
    cid = lax.axis_index("c")
    sid = lax.axis_index("s")
    wid = sid * NC + cid
    _k2_graph(head_sr, tail_sr, vals_sr, part_sr, adj_sr,
              d_sh, d_ref, rs, rowbuf, hb, tb, vb, ob, sem, cid, sid, wid)
    plsc.subcore_barrier()
    _k2_graph(head_tg, tail_tg, vals_tg, part_tg, adj_tg,
              d_sh, d_ref, rs, rowbuf, hb, tb, vb, ob, sem, cid, sid, wid)


@functools.cache
def _k2():
    return functools.partial(
        pl.kernel,
        out_type=(
            jax.ShapeDtypeStruct((E + N,), _f32),
            jax.ShapeDtypeStruct((E + N,), _f32),
        ),
        mesh=_mesh(),
        compiler_params=pltpu.CompilerParams(needs_layout_passes=False),
        scratch_types=(
            pltpu.VMEM_SHARED((NPAD,), _f32),
            pltpu.VMEM((NPAD,), _f32),
            pltpu.VMEM((SLICE,), _f32),
            pltpu.VMEM((SLICE,), _f32),
            pltpu.VMEM((B,), _i32),
            pltpu.VMEM((B,), _i32),
            pltpu.VMEM((B,), _f32),
            pltpu.VMEM((B,), _f32),
            pltpu.SemaphoreType.DMA,
        ),
    )(_k2_body)


def kernel(rel_emb_sr, rel_emb_tg, conf_sr, imp_sr, pca_sr,
           conf_tg, imp_tg, pca_tg, head_sr, tail_sr, relation_sr,
           head_tg, tail_tg, relation_tg):
    del rel_emb_sr, rel_emb_tg, relation_sr, relation_tg
    h_sr = head_sr.astype(_i32)
    t_sr = tail_sr.astype(_i32)
    h_tg = head_tg.astype(_i32)
    t_tg = tail_tg.astype(_i32)
    vals_sr, vals_tg, part_sr, part_tg = _k1()(
        conf_sr, imp_sr, pca_sr, h_sr, conf_tg, imp_tg, pca_tg, h_tg)
    adj_sr, adj_tg = _k2()(
        h_sr, t_sr, vals_sr, part_sr, h_tg, t_tg, vals_tg, part_tg)
    return adj_sr, adj_tg


# speedup vs baseline: 382.4749x; 2.0890x over previous
import functools

import jax
import jax.numpy as jnp
from jax import lax
from jax.experimental import pallas as pl
from jax.experimental.pallas import tpu as pltpu
from jax.experimental.pallas import tpu_sc as plsc

N = 50000
E = 1600000
NC = 2
NS = 16
NT = NC * NS
EPT = E // NT
B = 2000
NCH = EPT // B
VPC = B // 16
NPAD = 50176
SLICE = NPAD // NS
SV = SLICE // 16
DIAG_T = 25
DIAG_B = N // DIAG_T

_f32 = jnp.float32
_i32 = jnp.int32


@functools.cache
def _mesh():
    return plsc.VectorSubcoreMesh(
        core_axis_name="c", subcore_axis_name="s",
        num_cores=NC, num_subcores=NS)


def _rsqrt_newton(x):
    i = plsc.bitcast(x, _i32)
    y = plsc.bitcast(jnp.int32(0x5F3759DF) - (i >> 1), _f32)
    for _ in range(3):
        y = y * (1.5 - 0.5 * x * y * y)
    return jnp.where(x > 0.0, y, 0.0)


def _k1_graph(conf_h, imp_h, pca_h, head_h, vals_h, part_h,
              acc, cbufs, ibufs, pbufs, hbufs, vbufs, isems, osems, wid):
    base = wid * EPT

    def zero_body(i, c):
        for u in range(8):
            acc[pl.ds((i * 8 + u) * 16, 16)] = jnp.zeros((16,), _f32)
        return c
    lax.fori_loop(0, NPAD // 16 // 8, zero_body, 0)

    def in_descs(slot, k):
        off = base + k * B
        return (
            pltpu.make_async_copy(conf_h.at[pl.ds(off, B)], cbufs[slot], isems[slot]),
            pltpu.make_async_copy(imp_h.at[pl.ds(off, B)], ibufs[slot], isems[slot]),
            pltpu.make_async_copy(pca_h.at[pl.ds(off, B)], pbufs[slot], isems[slot]),
            pltpu.make_async_copy(head_h.at[pl.ds(off, B)], hbufs[slot], isems[slot]),
        )

    def issue_in(slot, k):
        for d in in_descs(slot, k):
            d.start()

    def wait_in(slot, k):
        for d in in_descs(slot, k):
            d.wait()

    def out_desc(slot, k):
        return pltpu.make_async_copy(
            vbufs[slot], vals_h.at[pl.ds(base + k * B, B)], osems[slot])

    def compute(slot):
        cb, ib, pb, hb, vb = (cbufs[slot], ibufs[slot], pbufs[slot],
                              hbufs[slot], vbufs[slot])

        def vec_body(i, c):
            for u in range(5):
                s = pl.ds((i * 5 + u) * 16, 16)
                v = cb[s] * ib[s] * pb[s]
                vb[s] = v
                plsc.addupdate_scatter(acc, [hb[s]], v)
            return c
        lax.fori_loop(0, VPC // 5, vec_body, 0)

    issue_in(0, 0)
    issue_in(1, 1)

    def outer(g, c):
        for b in range(2):
            k = 2 * g + b
            wait_in(b, k)

            @pl.when(k >= 2)
            def _():
                out_desc(b, k - 2).wait()
            compute(b)
            out_desc(b, k).start()

            @pl.when(k < NCH - 2)
            def _():
                issue_in(b, k + 2)
        return c
    lax.fori_loop(0, (NCH - 1) // 2, outer, 0)

    wait_in(0, NCH - 1)
    out_desc(0, NCH - 3).wait()
    compute(0)
    out_desc(0, NCH - 1).start()
    out_desc(1, NCH - 2).wait()
    out_desc(0, NCH - 1).wait()
    pltpu.sync_copy(acc, part_h.at[pl.ds(wid * NPAD, NPAD)])


def _k1_body(conf_sr, imp_sr, pca_sr, head_sr,
             conf_tg, imp_tg, pca_tg, head_tg,
             vals_sr, vals_tg, part_sr, part_tg,
             acc, cb0, cb1, ib0, ib1, pb0, pb1, hb0, hb1, vb0, vb1,
             isem0, isem1, osem0, osem1):
    wid = lax.axis_index("s") * NC + lax.axis_index("c")
    args = (acc, (cb0, cb1), (ib0, ib1), (pb0, pb1), (hb0, hb1), (vb0, vb1),
            (isem0, isem1), (osem0, osem1), wid)
    _k1_graph(conf_sr, imp_sr, pca_sr, head_sr, vals_sr, part_sr, *args)
    _k1_graph(conf_tg, imp_tg, pca_tg, head_tg, vals_tg, part_tg, *args)


@functools.cache
def _k1():
    return functools.partial(
        pl.kernel,
        out_type=(
            jax.ShapeDtypeStruct((E,), _f32),
            jax.ShapeDtypeStruct((E,), _f32),
            jax.ShapeDtypeStruct((NT * NPAD,), _f32),
            jax.ShapeDtypeStruct((NT * NPAD,), _f32),
        ),
        mesh=_mesh(),
        compiler_params=pltpu.CompilerParams(needs_layout_passes=False),
        scratch_types=(
            pltpu.VMEM((NPAD,), _f32),
            pltpu.VMEM((B,), _f32), pltpu.VMEM((B,), _f32),
            pltpu.VMEM((B,), _f32), pltpu.VMEM((B,), _f32),
            pltpu.VMEM((B,), _f32), pltpu.VMEM((B,), _f32),
            pltpu.VMEM((B,), _i32), pltpu.VMEM((B,), _i32),
            pltpu.VMEM((B,), _f32), pltpu.VMEM((B,), _f32),
            pltpu.SemaphoreType.DMA, pltpu.SemaphoreType.DMA,
            pltpu.SemaphoreType.DMA, pltpu.SemaphoreType.DMA,
        ),
    )(_k1_body)


def _k2_graph(head_h, tail_h, vals_h, part_h, adj_h,
              d_sh, d_ref, rs, rowbufs, hbufs, tbufs, vbufs, obufs,
              rsems, isems, osems, sid, wid):
    soff = sid * SLICE

    def one_body(i, c):
        for u in range(7):
            rs[pl.ds((i * 7 + u) * 16, 16)] = jnp.full((16,), 1.0, _f32)
        return c
    lax.fori_loop(0, SV // 7, one_body, 0)

    def row_desc(slot, r):
        return pltpu.make_async_copy(
            part_h.at[pl.ds(r * NPAD + soff, SLICE)], rowbufs[slot], rsems[slot])

    row_desc(0, 0).start()
    row_desc(1, 1).start()

    def row_outer(g, c):
        for b in range(2):
            r = 2 * g + b
            row_desc(b, r).wait()
            rb = rowbufs[b]

            def add_body(i, c2):
                for u in range(7):
                    s = pl.ds((i * 7 + u) * 16, 16)
                    rs[s] = rs[s] + rb[s]
                return c2
            lax.fori_loop(0, SV // 7, add_body, 0)

            @pl.when(r < NT - 2)
            def _():
                row_desc(b, r + 2).start()
        return c
    lax.fori_loop(0, NT // 2, row_outer, 0)

    def newton_body(i, c):
        for u in range(4):
            s = pl.ds((i * 4 + u) * 16, 16)
            rowbufs[0][s] = _rsqrt_newton(rs[s])
        return c
    lax.fori_loop(0, SV // 4, newton_body, 0)
    pltpu.sync_copy(rowbufs[0], d_sh.at[pl.ds(soff, SLICE)])
    plsc.subcore_barrier()

    pltpu.sync_copy(d_sh, d_ref)

    base = wid * EPT

    def in_descs(slot, k):
        off = base + k * B
        return (
            pltpu.make_async_copy(head_h.at[pl.ds(off, B)], hbufs[slot], isems[slot]),
            pltpu.make_async_copy(tail_h.at[pl.ds(off, B)], tbufs[slot], isems[slot]),
            pltpu.make_async_copy(vals_h.at[pl.ds(off, B)], vbufs[slot], isems[slot]),
        )

    def issue_in(slot, k):
        for d in in_descs(slot, k):
            d.start()

    def wait_in(slot, k):
        for d in in_descs(slot, k):
            d.wait()

    def out_desc(slot, k):
        return pltpu.make_async_copy(
            obufs[slot], adj_h.at[pl.ds(base + k * B, B)], osems[slot])

    def compute(slot):
        hb, tb, vb, ob = hbufs[slot], tbufs[slot], vbufs[slot], obufs[slot]

        def vec_body(i, c):
            for u in range(5):
                s = pl.ds((i * 5 + u) * 16, 16)
                dh = plsc.load_gather(d_ref, [hb[s]])
                dt = plsc.load_gather(d_ref, [tb[s]])
                ob[s] = vb[s] * dh * dt
            return c
        lax.fori_loop(0, VPC // 5, vec_body, 0)

    issue_in(0, 0)
    issue_in(1, 1)

    def outer(g, c):
        for b in range(2):
            k = 2 * g + b
            wait_in(b, k)

            @pl.when(k >= 2)
            def _():
                out_desc(b, k - 2).wait()
            compute(b)
            out_desc(b, k).start()

            @pl.when(k < NCH - 2)
            def _():
                issue_in(b, k + 2)
        return c
    lax.fori_loop(0, (NCH - 1) // 2, outer, 0)

    wait_in(0, NCH - 1)
    out_desc(0, NCH - 3).wait()
    compute(0)
    out_desc(0, NCH - 1).start()
    out_desc(1, NCH - 2).wait()
    out_desc(0, NCH - 1).wait()

    @pl.when(wid < DIAG_T)
    def _():
        doff = wid * DIAG_B
        ob = obufs[0]

        def diag_body(i, c):
            for u in range(5):
                j = i * 5 + u
                y = d_ref[pl.ds(doff + j * 16, 16)]
                ob[pl.ds(j * 16, 16)] = y * y
            return c
        lax.fori_loop(0, DIAG_B // 16 // 5, diag_body, 0)
        pltpu.sync_copy(ob, adj_h.at[pl.ds(E + doff, DIAG_B)])


def _k2_body(head_sr, tail_sr, vals_sr, part_sr,
             head_tg, tail_tg, vals_tg, part_tg,
             adj_sr, adj_tg,
             d_sh, d_ref, rs, rb0, rb1, hb0, hb1, tb0, tb1, vb0, vb1,
             ob0, ob1, rsem0, rsem1, isem0, isem1, osem0, osem1):
    sid = lax.axis_index("s")
    wid = sid * NC + lax.axis_index("c")
    args = (d_sh, d_ref, rs, (rb0, rb1), (hb0, hb1), (tb0, tb1), (vb0, vb1),
            (ob0, ob1), (rsem0, rsem1), (isem0, isem1), (osem0, osem1),
            sid, wid)
    _k2_graph(head_sr, tail_sr, vals_sr, part_sr, adj_sr, *args)
    plsc.subcore_barrier()
    _k2_graph(head_tg, tail_tg, vals_tg, part_tg, adj_tg, *args)


@functools.cache
def _k2():
    return functools.partial(
        pl.kernel,
        out_type=(
            jax.ShapeDtypeStruct((E + N,), _f32),
            jax.ShapeDtypeStruct((E + N,), _f32),
        ),
        mesh=_mesh(),
        compiler_params=pltpu.CompilerParams(needs_layout_passes=False),
        scratch_types=(
            pltpu.VMEM_SHARED((NPAD,), _f32),
            pltpu.VMEM((NPAD,), _f32),
            pltpu.VMEM((SLICE,), _f32),
            pltpu.VMEM((SLICE,), _f32), pltpu.VMEM((SLICE,), _f32),
            pltpu.VMEM((B,), _i32), pltpu.VMEM((B,), _i32),
            pltpu.VMEM((B,), _i32), pltpu.VMEM((B,), _i32),
            pltpu.VMEM((B,), _f32), pltpu.VMEM((B,), _f32),
            pltpu.VMEM((B,), _f32), pltpu.VMEM((B,), _f32),
            pltpu.SemaphoreType.DMA, pltpu.SemaphoreType.DMA,
            pltpu.SemaphoreType.DMA, pltpu.SemaphoreType.DMA,
            pltpu.SemaphoreType.DMA, pltpu.SemaphoreType.DMA,
        ),
    )(_k2_body)


def kernel(rel_emb_sr, rel_emb_tg, conf_sr, imp_sr, pca_sr,
           conf_tg, imp_tg, pca_tg, head_sr, tail_sr, relation_sr,
           head_tg, tail_tg, relation_tg):
    del rel_emb_sr, rel_emb_tg, relation_sr, relation_tg
    h_sr = head_sr.astype(_i32)
    t_sr = tail_sr.astype(_i32)
    h_tg = head_tg.astype(_i32)
    t_tg = tail_tg.astype(_i32)
    vals_sr, vals_tg, part_sr, part_tg = _k1()(
        conf_sr, imp_sr, pca_sr, h_sr, conf_tg, imp_tg, pca_tg, h_tg)
    adj_sr, adj_tg = _k2()(
        h_sr, t_sr, vals_sr, part_sr, h_tg, t_tg, vals_tg, part_tg)
    return adj_sr, adj_tg
